# Initial kernel scaffold; baseline (speedup 1.0000x reference)
import jax
import jax.numpy as jnp
from jax.experimental import pallas as pl


def kernel(x, edge_index, W, b):
    raise NotImplementedError("write your pallas kernel here")


# speedup vs baseline: 12.4868x; 12.4868x over previous
import functools

import jax
import jax.numpy as jnp
from jax import lax
from jax.experimental import pallas as pl
from jax.experimental.pallas import tpu as pltpu
from jax.experimental.pallas import tpu_sc as plsc

f32 = jnp.float32

N = 10000
D = 256
E = 160000
DH = D // 2

NC, NS, LANES = 2, 16, 16
NW = NC * NS
CH = 128
CPW = -(-E // (NW * CH))
EPAD = NW * CPW * CH
CPT = -(-E // (NS * CH))
EPADC = NS * CPT * CH

NR = -(-(N + 1) // (NS * CH)) * (NS * CH)
PT = NR // NS
ZC = PT // CH

_mesh = plsc.VectorSubcoreMesh(
    core_axis_name="c", subcore_axis_name="s", num_cores=NC, num_subcores=NS)


def _fill(ref, n, value):
  def body(i, _):
    ref[pl.ds(i * 16, 16)] = jnp.full((16,), value, f32)
    return 0
  lax.fori_loop(0, n // 16, body, 0)


@functools.partial(
    pl.kernel,
    out_type=jax.ShapeDtypeStruct((NC * NR,), f32),
    mesh=_mesh,
    scratch_types=[
        pltpu.VMEM((CPW, CH), jnp.int32),
        pltpu.VMEM((CH,), f32),
        pltpu.VMEM((PT,), f32),
        pltpu.VMEM_SHARED((NR,), f32),
    ],
)
def _deg_kernel(dstp_ref, deg_ref, idx_v, ones_v, zbuf, deg_s):
  c = lax.axis_index("c")
  s = lax.axis_index("s")
  wid = c * NS + s
  _fill(zbuf, PT, 0.0)
  _fill(ones_v, CH, 1.0)
  pltpu.sync_copy(zbuf, deg_s.at[pl.ds(s * PT, PT)])
  plsc.subcore_barrier()
  pltpu.sync_copy(dstp_ref.at[wid], idx_v)

  def body(j, _):
    pltpu.sync_copy(ones_v, deg_s.at[idx_v.at[j]], add=True)
    return 0
  lax.fori_loop(0, CPW, body, 0)
  plsc.subcore_barrier()
  pltpu.sync_copy(deg_s.at[pl.ds(s * PT, PT)],
                  deg_ref.at[pl.ds(c * NR + s * PT, PT)])


@functools.partial(
    pl.kernel,
    out_type=jax.ShapeDtypeStruct((NC * NR, DH), f32),
    mesh=_mesh,
    scratch_types=[
        pltpu.VMEM((CPT, CH), jnp.int32),
        pltpu.VMEM((CPT, CH), jnp.int32),
        pltpu.VMEM((CH, DH), f32),
        pltpu.VMEM_SHARED((NR, DH), f32),
    ],
)
def _scatter_kernel(h2_ref, srcp_ref, dstp_ref, s_ref, isrc, idst, rows, acc):
  c = lax.axis_index("c")
  s = lax.axis_index("s")
  wid = c * NS + s
  base = s * PT

  def zrow(r, _):
    for k in range(DH // 16):
      rows[r, pl.ds(k * 16, 16)] = jnp.zeros((16,), f32)
    return 0
  lax.fori_loop(0, CH, zrow, 0)
  for t in range(ZC):
    pltpu.sync_copy(rows, acc.at[pl.ds(base + t * CH, CH)])
  plsc.subcore_barrier()

  pltpu.sync_copy(srcp_ref.at[wid], isrc)
  pltpu.sync_copy(dstp_ref.at[wid], idst)

  def body(j, _):
    pltpu.sync_copy(h2_ref.at[isrc.at[j]], rows)
    pltpu.sync_copy(rows, acc.at[idst.at[j]], add=True)
    return 0
  lax.fori_loop(0, CPT, body, 0)

  plsc.subcore_barrier()
  for t in range(ZC):
    pltpu.sync_copy(acc.at[pl.ds(base + t * CH, CH)],
                    s_ref.at[pl.ds(c * NR + base + t * CH, CH)])


_R = 1000


def _tc_mm(x, w, deg0, deg1):
  def body(x_ref, w_ref, d0_ref, d1_ref, h2_ref, dinv_ref):
    deg = d0_ref[...] + d1_ref[...] + 1.0
    dinv = lax.rsqrt(deg)
    h = jnp.dot(x_ref[...], w_ref[...], preferred_element_type=f32) * dinv
    h2_ref[0] = h[:, :DH]
    h2_ref[1] = h[:, DH:]
    dinv_ref[...] = dinv

  return pl.pallas_call(
      body,
      grid=(N // _R,),
      in_specs=[
          pl.BlockSpec((_R, D), lambda i: (i, 0)),
          pl.BlockSpec((D, D), lambda i: (0, 0)),
          pl.BlockSpec((_R, 1), lambda i: (i, 0)),
          pl.BlockSpec((_R, 1), lambda i: (i, 0)),
      ],
      out_specs=[
          pl.BlockSpec((2, _R, DH), lambda i: (0, i, 0)),
          pl.BlockSpec((_R, 1), lambda i: (i, 0)),
      ],
      out_shape=[
          jax.ShapeDtypeStruct((2, N, DH), f32),
          jax.ShapeDtypeStruct((N, 1), f32),
      ],
  )(x, w, deg0, deg1)


def _tc_out(x, h0, h1, s0, s1, dinv, b):
  inv_sqrt2 = 0.7071067811865476

  def body(x_ref, h0_ref, h1_ref, s0_ref, s1_ref, dinv_ref, b_ref, o_ref):
    dinv = dinv_ref[...]
    z0 = dinv * (s0_ref[...] + h0_ref[...]) + b_ref[:, :DH]
    z1 = dinv * (s1_ref[...] + h1_ref[...]) + b_ref[:, DH:]
    z = jnp.concatenate([z0, z1], axis=1)
    g = 0.5 * z * (1.0 + lax.erf(z * inv_sqrt2))
    o_ref[...] = x_ref[...] + g

  return pl.pallas_call(
      body,
      grid=(N // _R,),
      in_specs=[
          pl.BlockSpec((_R, D), lambda i: (i, 0)),
          pl.BlockSpec((_R, DH), lambda i: (i, 0)),
          pl.BlockSpec((_R, DH), lambda i: (i, 0)),
          pl.BlockSpec((_R, DH), lambda i: (i, 0)),
          pl.BlockSpec((_R, DH), lambda i: (i, 0)),
          pl.BlockSpec((_R, 1), lambda i: (i, 0)),
          pl.BlockSpec((1, D), lambda i: (0, 0)),
      ],
      out_specs=pl.BlockSpec((_R, D), lambda i: (i, 0)),
      out_shape=jax.ShapeDtypeStruct((N, D), f32),
  )(x, h0, h1, s0, s1, dinv, b)


def kernel(x, edge_index, W, b):
  src = edge_index[0]
  dst = edge_index[1]
  pad = EPAD - E
  dstp_deg = jnp.concatenate([dst, jnp.full((pad,), N, jnp.int32)]).reshape(
      NW, CPW, CH)
  padc = EPADC - E
  srcc = jnp.concatenate([src, jnp.zeros((padc,), jnp.int32)]).reshape(
      NS, CPT, CH)
  dstc = jnp.concatenate([dst, jnp.full((padc,), N, jnp.int32)]).reshape(
      NS, CPT, CH)
  srcp = jnp.concatenate([srcc, srcc + N]).reshape(NW, CPT, CH)
  dstp = jnp.concatenate([dstc, dstc]).reshape(NW, CPT, CH)

  deg2 = _deg_kernel(dstp_deg)
  h2, dinv = _tc_mm(x, W, deg2[:N].reshape(N, 1),
                    deg2[NR:NR + N].reshape(N, 1))
  s2 = _scatter_kernel(h2.reshape(2 * N, DH), srcp, dstp)
  return _tc_out(x, h2[0], h2[1], s2[:N], s2[NR:NR + N], dinv,
                 b.reshape(1, D))
